# baseline (speedup 1.0000x reference)
import functools

import jax
import jax.numpy as jnp
from jax import lax
from jax.experimental import pallas as pl
from jax.experimental.pallas import tpu as pltpu
from jax.experimental.pallas import tpu_sc as plsc

_NUM_HEADS = 8
_N_PER_HEAD = 100000
_TOTAL_N = _NUM_HEADS * _N_PER_HEAD
_D = 32
_B_ROWS = 16384

_NC = 2
_NS = 16
_L = 16
_NJ = 783
_NQ = _B_ROWS // 4
_NGQ = _NQ // _L


def _emb_body(ids_hbm, p4, out_hbm,
              idx_v, buf_v, rows_a, rows_b, sem_w, sem_i, sem_a, sem_b):
    wid = lax.axis_index("s") * _NC + lax.axis_index("c")
    i = lax.shift_right_logical(wid, jnp.int32(3))
    k = wid & jnp.int32(7)

    rows = (rows_a, rows_b)
    sems = (sem_a, sem_b)
    pending = [None, None]

    for h in range(_NUM_HEADS):
        j0 = min((h * _N_PER_HEAD) // 128, 6250 - _NJ)
        shift = h * _N_PER_HEAD - j0 * 128
        hw = pltpu.async_copy(p4.at[i, pl.ds(j0, _NJ), k], buf_v, sem_w)
        hi = pltpu.async_copy(ids_hbm.at[h], idx_v, sem_i)
        hw.wait()
        hi.wait()

        for q in range(4):
            par = q % 2
            if pending[par] is not None:
                pending[par].wait()

            def _extract(j, carry):
                a = idx_v[pl.ds((q * _NGQ + j) * _L, _L)] + jnp.int32(shift)
                a = jnp.minimum(
                    jnp.maximum(a, jnp.int32(0)), jnp.int32(_NJ * 128 - 1)
                )
                jv = lax.shift_right_logical(a, jnp.int32(7))
                lv = a & jnp.int32(127)
                rows[par][pl.ds(j * _L, _L)] = plsc.load_gather(buf_v, [jv, lv])
                return carry

            lax.fori_loop(0, _NGQ, _extract, 0, unroll=8)
            pending[par] = pltpu.async_copy(
                rows[par], out_hbm.at[h, wid, pl.ds(q * _NQ, _NQ)], sems[par]
            )

    pending[0].wait()
    pending[1].wait()


@jax.jit
def kernel(input_ids, table):
    p4 = table.reshape(_TOTAL_N // 128, 128, 4, 8).transpose(2, 0, 3, 1)
    gather = functools.partial(
        pl.kernel,
        out_type=jax.ShapeDtypeStruct((_NUM_HEADS, _D, _B_ROWS), jnp.float32),
        mesh=plsc.VectorSubcoreMesh(core_axis_name="c", subcore_axis_name="s"),
        scratch_types=[
            pltpu.VMEM((_B_ROWS,), jnp.int32),
            pltpu.VMEM((_NJ, 128), jnp.float32),
            pltpu.VMEM((_NQ,), jnp.float32),
            pltpu.VMEM((_NQ,), jnp.float32),
            pltpu.SemaphoreType.DMA,
            pltpu.SemaphoreType.DMA,
            pltpu.SemaphoreType.DMA,
            pltpu.SemaphoreType.DMA,
        ],
        compiler_params=pltpu.CompilerParams(
            use_tc_tiling_on_sc=False, needs_layout_passes=False
        ),
    )(_emb_body)
    out3 = gather(input_ids.T, p4)
    return out3.transpose(2, 0, 1)
